# baseline (speedup 1.0000x reference)
import functools

import jax
import jax.numpy as jnp
from jax import lax
from jax.experimental import pallas as pl
from jax.experimental.pallas import tpu as pltpu
from jax.experimental.pallas import tpu_sc as plsc

NUM_EXPERTS = 16
TOPK = 2
DIM = 2048
INTER = 1664
T = 4096
RP = T * TOPK

BM = 256
NUM_TILES = RP // BM
MAX_STEPS = NUM_TILES + NUM_EXPERTS - 1

NC = 2
NS = 16
NW = NC * NS
G_ROWS_PER_W = RP // NW
G_CH = 16
G_NCH = G_ROWS_PER_W // G_CH
C_TOK_PER_W = T // NW
C_CH = 8
C_NCH = C_TOK_PER_W // C_CH


def _sc_gather(idx, table):
    mesh = plsc.VectorSubcoreMesh(core_axis_name="c", subcore_axis_name="s")

    @functools.partial(
        pl.kernel,
        mesh=mesh,
        out_type=jax.ShapeDtypeStruct((RP, DIM // 2), jnp.int32),
        scratch_types=[
            pltpu.VMEM((G_ROWS_PER_W,), jnp.int32),
            pltpu.VMEM((2, G_CH, DIM // 2), jnp.int32),
            pltpu.SemaphoreType.DMA,
            pltpu.SemaphoreType.DMA,
            pltpu.SemaphoreType.DMA,
            pltpu.SemaphoreType.DMA,
        ],
    )
    def gather_kernel(idx_hbm, table_hbm, out_hbm, idx_v, rows_v,
                      gs0, gs1, ws0, ws1):
        wid = lax.axis_index("s") * NC + lax.axis_index("c")
        base = wid * G_ROWS_PER_W
        pltpu.sync_copy(idx_hbm.at[pl.ds(base, G_ROWS_PER_W)], idx_v)
        gsems = (gs0, gs1)
        wsems = (ws0, ws1)
        gh = [None, None]
        wh = [None, None]
        for c in range(G_NCH):
            s = c & 1
            if c >= 2:
                wh[s].wait()
            ids = idx_v[pl.ds(c * G_CH, G_CH)]
            gh[s] = pltpu.async_copy(table_hbm.at[ids], rows_v.at[s], gsems[s])
            if c >= 1:
                p = (c - 1) & 1
                gh[p].wait()
                wh[p] = pltpu.async_copy(
                    rows_v.at[p],
                    out_hbm.at[pl.ds(base + (c - 1) * G_CH, G_CH)], wsems[p])
        lastp = (G_NCH - 1) & 1
        gh[lastp].wait()
        wh[lastp] = pltpu.async_copy(
            rows_v.at[lastp],
            out_hbm.at[pl.ds(base + (G_NCH - 1) * G_CH, G_CH)], wsems[lastp])
        wh[1 - lastp].wait()
        wh[lastp].wait()

    return gather_kernel(idx, table)


def _sc_combine(p0, p1, rows):
    mesh = plsc.VectorSubcoreMesh(core_axis_name="c", subcore_axis_name="s")

    @functools.partial(
        pl.kernel,
        mesh=mesh,
        out_type=jax.ShapeDtypeStruct((T, DIM), jnp.float32),
        scratch_types=[
            pltpu.VMEM((C_NCH, C_CH), jnp.int32),
            pltpu.VMEM((C_NCH, C_CH), jnp.int32),
            pltpu.VMEM((2, C_CH, DIM), jnp.float32),
            pltpu.VMEM((2, C_CH, DIM), jnp.float32),
            pltpu.SemaphoreType.DMA,
            pltpu.SemaphoreType.DMA,
            pltpu.SemaphoreType.DMA,
            pltpu.SemaphoreType.DMA,
            pltpu.SemaphoreType.DMA,
            pltpu.SemaphoreType.DMA,
        ],
    )
    def combine_kernel(p0_hbm, p1_hbm, rows_hbm, out_hbm, p0_v, p1_v,
                       bufa, bufb, ga0, ga1, gb0, gb1, ws0, ws1):
        wid = lax.axis_index("s") * NC + lax.axis_index("c")
        base = wid * C_TOK_PER_W
        cbase = wid * C_NCH
        pltpu.sync_copy(p0_hbm.at[pl.ds(cbase, C_NCH)], p0_v)
        pltpu.sync_copy(p1_hbm.at[pl.ds(cbase, C_NCH)], p1_v)
        gasems = (ga0, ga1)
        gbsems = (gb0, gb1)
        wsems = (ws0, ws1)
        gha = [None, None]
        ghb = [None, None]
        wh = [None, None]

        def add_and_write(p, c):
            for r in range(C_CH):
                def add_body(l, cc, _r=r, _p=p):
                    for u in range(8):
                        o = (l * 8 + u) * 16
                        bufa[_p, _r, pl.ds(o, 16)] = (
                            bufa[_p, _r, pl.ds(o, 16)]
                            + bufb[_p, _r, pl.ds(o, 16)])
                    return cc
                lax.fori_loop(0, DIM // (8 * 16), add_body, 0)
            return pltpu.async_copy(
                bufa.at[p], out_hbm.at[pl.ds(base + c * C_CH, C_CH)], wsems[p])

        for c in range(C_NCH):
            s = c & 1
            if c >= 2:
                wh[s].wait()
            gha[s] = pltpu.async_copy(rows_hbm.at[p0_v.at[c]], bufa.at[s],
                                      gasems[s])
            ghb[s] = pltpu.async_copy(rows_hbm.at[p1_v.at[c]], bufb.at[s],
                                      gbsems[s])
            if c >= 1:
                p = (c - 1) & 1
                gha[p].wait()
                ghb[p].wait()
                wh[p] = add_and_write(p, c - 1)
        lastp = (C_NCH - 1) & 1
        gha[lastp].wait()
        ghb[lastp].wait()
        wh[lastp] = add_and_write(lastp, C_NCH - 1)
        wh[1 - lastp].wait()
        wh[lastp].wait()

    return combine_kernel(p0, p1, rows)


def _ffn1_body(tid_ref, gid_ref, rs_ref, re_ref, x_ref, w1_ref, w3_ref, out_ref):
    i = pl.program_id(0)
    dn = (((1,), (1,)), ((), ()))
    xb = x_ref[...].astype(jnp.bfloat16)
    a1 = lax.dot_general(xb, w1_ref[0], dn, preferred_element_type=jnp.float32)
    a3 = lax.dot_general(xb, w3_ref[0], dn, preferred_element_type=jnp.float32)
    h = (a1 * lax.logistic(a1) * a3).astype(jnp.bfloat16)
    rows = tid_ref[i] * BM + lax.broadcasted_iota(jnp.int32, (BM, 1), 0)
    mask = (rows >= rs_ref[i]) & (rows < re_ref[i])
    first = jnp.logical_or(i == 0, tid_ref[i] != tid_ref[jnp.maximum(i - 1, 0)])

    @pl.when(first)
    def _():
        out_ref[...] = jnp.where(mask, h, jnp.zeros_like(h))

    @pl.when(jnp.logical_not(first))
    def _():
        out_ref[...] = jnp.where(mask, h, out_ref[...])


def _ffn2_body(tid_ref, gid_ref, rs_ref, re_ref, h_ref, w2_ref, ws_ref, out_ref):
    i = pl.program_id(0)
    dn = (((1,), (1,)), ((), ()))
    a = lax.dot_general(h_ref[...], w2_ref[0], dn, preferred_element_type=jnp.float32)
    a = a * ws_ref[...]
    rows = tid_ref[i] * BM + lax.broadcasted_iota(jnp.int32, (BM, 1), 0)
    mask = (rows >= rs_ref[i]) & (rows < re_ref[i])
    first = jnp.logical_or(i == 0, tid_ref[i] != tid_ref[jnp.maximum(i - 1, 0)])

    @pl.when(first)
    def _():
        out_ref[...] = jnp.where(mask, a, jnp.zeros_like(a))

    @pl.when(jnp.logical_not(first))
    def _():
        out_ref[...] = jnp.where(mask, a, out_ref[...])


def _ffn_fused_body(tid_ref, gid_ref, rs_ref, re_ref, x_ref, w1_ref, w3_ref,
                    w2_ref, ws_ref, out_ref):
    i = pl.program_id(0)
    dn = (((1,), (1,)), ((), ()))
    xb = x_ref[...]
    a1 = lax.dot_general(xb, w1_ref[0], dn, preferred_element_type=jnp.float32)
    a3 = lax.dot_general(xb, w3_ref[0], dn, preferred_element_type=jnp.float32)
    h = (a1 * lax.logistic(a1) * a3).astype(jnp.bfloat16)
    d = lax.dot_general(h, w2_ref[0], dn, preferred_element_type=jnp.float32)
    d = d * ws_ref[...]
    rows = tid_ref[i] * BM + lax.broadcasted_iota(jnp.int32, (BM, 1), 0)
    mask = (rows >= rs_ref[i]) & (rows < re_ref[i])
    first = jnp.logical_or(i == 0, tid_ref[i] != tid_ref[jnp.maximum(i - 1, 0)])

    @pl.when(first)
    def _():
        out_ref[...] = jnp.where(mask, d, jnp.zeros_like(d))

    @pl.when(jnp.logical_not(first))
    def _():
        out_ref[...] = jnp.where(mask, d, out_ref[...])


def _grouped_ffn_fused(tid, gid, rs, re, xp, w1, w3, w2, ws):
    grid_spec = pltpu.PrefetchScalarGridSpec(
        num_scalar_prefetch=4,
        grid=(MAX_STEPS,),
        in_specs=[
            pl.BlockSpec((BM, DIM), lambda i, t, g, s, e: (t[i], 0)),
            pl.BlockSpec((1, INTER, DIM), lambda i, t, g, s, e: (g[i], 0, 0)),
            pl.BlockSpec((1, INTER, DIM), lambda i, t, g, s, e: (g[i], 0, 0)),
            pl.BlockSpec((1, DIM, INTER), lambda i, t, g, s, e: (g[i], 0, 0)),
            pl.BlockSpec((BM, 1), lambda i, t, g, s, e: (t[i], 0)),
        ],
        out_specs=pl.BlockSpec((BM, DIM), lambda i, t, g, s, e: (t[i], 0)),
    )
    return pl.pallas_call(
        _ffn_fused_body,
        grid_spec=grid_spec,
        out_shape=jax.ShapeDtypeStruct((RP, DIM), jnp.float32),
        compiler_params=pltpu.CompilerParams(
            dimension_semantics=("arbitrary",)),
    )(tid, gid, rs, re, xp, w1, w3, w2, ws)


def _grouped_ffn1(tid, gid, rs, re, xp, w1, w3):
    grid_spec = pltpu.PrefetchScalarGridSpec(
        num_scalar_prefetch=4,
        grid=(MAX_STEPS,),
        in_specs=[
            pl.BlockSpec((BM, DIM), lambda i, t, g, s, e: (t[i], 0)),
            pl.BlockSpec((1, INTER, DIM), lambda i, t, g, s, e: (g[i], 0, 0)),
            pl.BlockSpec((1, INTER, DIM), lambda i, t, g, s, e: (g[i], 0, 0)),
        ],
        out_specs=pl.BlockSpec((BM, INTER), lambda i, t, g, s, e: (t[i], 0)),
    )
    return pl.pallas_call(
        _ffn1_body,
        grid_spec=grid_spec,
        out_shape=jax.ShapeDtypeStruct((RP, INTER), jnp.bfloat16),
        compiler_params=pltpu.CompilerParams(
            dimension_semantics=("arbitrary",)),
    )(tid, gid, rs, re, xp, w1, w3)


def _grouped_ffn2(tid, gid, rs, re, up, w2, ws):
    grid_spec = pltpu.PrefetchScalarGridSpec(
        num_scalar_prefetch=4,
        grid=(MAX_STEPS,),
        in_specs=[
            pl.BlockSpec((BM, INTER), lambda i, t, g, s, e: (t[i], 0)),
            pl.BlockSpec((1, DIM, INTER), lambda i, t, g, s, e: (g[i], 0, 0)),
            pl.BlockSpec((BM, 1), lambda i, t, g, s, e: (t[i], 0)),
        ],
        out_specs=pl.BlockSpec((BM, DIM), lambda i, t, g, s, e: (t[i], 0)),
    )
    return pl.pallas_call(
        _ffn2_body,
        grid_spec=grid_spec,
        out_shape=jax.ShapeDtypeStruct((RP, DIM), jnp.float32),
        compiler_params=pltpu.CompilerParams(
            dimension_semantics=("arbitrary",)),
    )(tid, gid, rs, re, up, w2, ws)


def _routing_metadata(expert_indices):
    flat = expert_indices.reshape(-1).astype(jnp.int32)
    sorted_idx = jnp.argsort(flat).astype(jnp.int32)
    gather_idx = sorted_idx // TOPK
    counts = jnp.bincount(flat, length=NUM_EXPERTS).astype(jnp.int32)
    ends = jnp.cumsum(counts)
    starts = ends - counts
    t_lo = starts // BM
    t_hi = (ends + BM - 1) // BM
    gt = jnp.where(counts > 0, t_hi - t_lo, 0)
    steps = jnp.arange(MAX_STEPS, dtype=jnp.int32)
    gid = jnp.repeat(jnp.arange(NUM_EXPERTS, dtype=jnp.int32), gt,
                     total_repeat_length=MAX_STEPS)
    cum_gt = jnp.cumsum(gt) - gt
    num_steps = jnp.sum(gt)
    tid_raw = t_lo[gid] + steps - cum_gt[gid]
    valid = steps < num_steps
    tid = jnp.where(valid, tid_raw, NUM_TILES - 1).astype(jnp.int32)
    rs = jnp.where(valid, jnp.maximum(starts[gid], tid_raw * BM), 0).astype(jnp.int32)
    re = jnp.where(valid, jnp.minimum(ends[gid], (tid_raw + 1) * BM), 0).astype(jnp.int32)
    inv = jnp.zeros((RP,), jnp.int32).at[sorted_idx].set(
        jnp.arange(RP, dtype=jnp.int32))
    p0 = inv[0::2]
    p1 = inv[1::2]
    return sorted_idx, gather_idx, tid, gid, rs, re, p0, p1


def kernel(x, expert_indices, expert_weights, w1, w2, w3):
    sorted_idx, gather_idx, tid, gid, rs, re, p0, p1 = _routing_metadata(
        expert_indices)
    ws_sorted = expert_weights.reshape(-1)[sorted_idx].reshape(RP, 1)

    w1b = w1.astype(jnp.bfloat16)
    w3b = w3.astype(jnp.bfloat16)
    w2b = w2.astype(jnp.bfloat16)

    x32 = lax.bitcast_convert_type(
        x.astype(jnp.bfloat16).reshape(T, DIM // 2, 2), jnp.int32)
    xp32 = _sc_gather(gather_idx, x32)
    xp = lax.bitcast_convert_type(xp32, jnp.bfloat16).reshape(RP, DIM)
    down = _grouped_ffn_fused(tid, gid, rs, re, xp, w1b, w3b, w2b, ws_sorted)
    out = _sc_combine(p0.reshape(T // C_CH, C_CH),
                      p1.reshape(T // C_CH, C_CH), down)
    return out


# speedup vs baseline: 1.4656x; 1.4656x over previous
import functools

import jax
import jax.numpy as jnp
from jax import lax
from jax.experimental import pallas as pl
from jax.experimental.pallas import tpu as pltpu
from jax.experimental.pallas import tpu_sc as plsc

NUM_EXPERTS = 16
TOPK = 2
DIM = 2048
INTER = 1664
T = 4096
RP = T * TOPK

BM = 256
NUM_TILES = RP // BM
MAX_STEPS = NUM_TILES + NUM_EXPERTS - 1

NC = 2
NS = 16
NW = NC * NS
G_ROWS_PER_W = RP // NW
G_CH = 16
G_NCH = G_ROWS_PER_W // G_CH
C_TOK_PER_W = T // NW
C_CH = 8
C_NCH = C_TOK_PER_W // C_CH


def _sc_gather(idx, table):
    mesh = plsc.VectorSubcoreMesh(core_axis_name="c", subcore_axis_name="s")

    @functools.partial(
        pl.kernel,
        mesh=mesh,
        out_type=jax.ShapeDtypeStruct((RP, DIM), jnp.float32),
        scratch_types=[
            pltpu.VMEM((G_ROWS_PER_W,), jnp.int32),
            pltpu.VMEM((2, G_CH, DIM), jnp.float32),
            pltpu.SemaphoreType.DMA,
            pltpu.SemaphoreType.DMA,
            pltpu.SemaphoreType.DMA,
            pltpu.SemaphoreType.DMA,
        ],
    )
    def gather_kernel(idx_hbm, table_hbm, out_hbm, idx_v, rows_v,
                      gs0, gs1, ws0, ws1):
        wid = lax.axis_index("s") * NC + lax.axis_index("c")
        base = wid * G_ROWS_PER_W
        pltpu.sync_copy(idx_hbm.at[pl.ds(base, G_ROWS_PER_W)], idx_v)
        gsems = (gs0, gs1)
        wsems = (ws0, ws1)
        gh = [None, None]
        wh = [None, None]
        for c in range(G_NCH):
            s = c & 1
            if c >= 2:
                wh[s].wait()
            ids = idx_v[pl.ds(c * G_CH, G_CH)]
            gh[s] = pltpu.async_copy(table_hbm.at[ids], rows_v.at[s], gsems[s])
            if c >= 1:
                p = (c - 1) & 1
                gh[p].wait()
                wh[p] = pltpu.async_copy(
                    rows_v.at[p],
                    out_hbm.at[pl.ds(base + (c - 1) * G_CH, G_CH)], wsems[p])
        lastp = (G_NCH - 1) & 1
        gh[lastp].wait()
        wh[lastp] = pltpu.async_copy(
            rows_v.at[lastp],
            out_hbm.at[pl.ds(base + (G_NCH - 1) * G_CH, G_CH)], wsems[lastp])
        wh[1 - lastp].wait()
        wh[lastp].wait()

    return gather_kernel(idx, table)


def _sc_combine(p0, p1, rows):
    mesh = plsc.VectorSubcoreMesh(core_axis_name="c", subcore_axis_name="s")

    @functools.partial(
        pl.kernel,
        mesh=mesh,
        out_type=jax.ShapeDtypeStruct((T, DIM), jnp.float32),
        scratch_types=[
            pltpu.VMEM((C_NCH, C_CH), jnp.int32),
            pltpu.VMEM((C_NCH, C_CH), jnp.int32),
            pltpu.VMEM((2, C_CH, DIM), jnp.float32),
            pltpu.VMEM((2, C_CH, DIM), jnp.float32),
            pltpu.SemaphoreType.DMA,
            pltpu.SemaphoreType.DMA,
            pltpu.SemaphoreType.DMA,
            pltpu.SemaphoreType.DMA,
            pltpu.SemaphoreType.DMA,
            pltpu.SemaphoreType.DMA,
        ],
    )
    def combine_kernel(p0_hbm, p1_hbm, rows_hbm, out_hbm, p0_v, p1_v,
                       bufa, bufb, ga0, ga1, gb0, gb1, ws0, ws1):
        wid = lax.axis_index("s") * NC + lax.axis_index("c")
        base = wid * C_TOK_PER_W
        cbase = wid * C_NCH
        pltpu.sync_copy(p0_hbm.at[pl.ds(cbase, C_NCH)], p0_v)
        pltpu.sync_copy(p1_hbm.at[pl.ds(cbase, C_NCH)], p1_v)
        gasems = (ga0, ga1)
        gbsems = (gb0, gb1)
        wsems = (ws0, ws1)
        gha = [None, None]
        ghb = [None, None]
        wh = [None, None]

        def add_and_write(p, c):
            for r in range(C_CH):
                def add_body(l, cc, _r=r, _p=p):
                    for u in range(8):
                        o = (l * 8 + u) * 16
                        bufa[_p, _r, pl.ds(o, 16)] = (
                            bufa[_p, _r, pl.ds(o, 16)]
                            + bufb[_p, _r, pl.ds(o, 16)])
                    return cc
                lax.fori_loop(0, DIM // (8 * 16), add_body, 0)
            return pltpu.async_copy(
                bufa.at[p], out_hbm.at[pl.ds(base + c * C_CH, C_CH)], wsems[p])

        for c in range(C_NCH):
            s = c & 1
            if c >= 2:
                wh[s].wait()
            gha[s] = pltpu.async_copy(rows_hbm.at[p0_v.at[c]], bufa.at[s],
                                      gasems[s])
            ghb[s] = pltpu.async_copy(rows_hbm.at[p1_v.at[c]], bufb.at[s],
                                      gbsems[s])
            if c >= 1:
                p = (c - 1) & 1
                gha[p].wait()
                ghb[p].wait()
                wh[p] = add_and_write(p, c - 1)
        lastp = (C_NCH - 1) & 1
        gha[lastp].wait()
        ghb[lastp].wait()
        wh[lastp] = add_and_write(lastp, C_NCH - 1)
        wh[1 - lastp].wait()
        wh[lastp].wait()

    return combine_kernel(p0, p1, rows)


def _ffn1_body(tid_ref, gid_ref, rs_ref, re_ref, x_ref, w1_ref, w3_ref, out_ref):
    i = pl.program_id(0)
    dn = (((1,), (1,)), ((), ()))
    xb = x_ref[...].astype(jnp.bfloat16)
    a1 = lax.dot_general(xb, w1_ref[0], dn, preferred_element_type=jnp.float32)
    a3 = lax.dot_general(xb, w3_ref[0], dn, preferred_element_type=jnp.float32)
    h = (a1 * lax.logistic(a1) * a3).astype(jnp.bfloat16)
    rows = tid_ref[i] * BM + lax.broadcasted_iota(jnp.int32, (BM, 1), 0)
    mask = (rows >= rs_ref[i]) & (rows < re_ref[i])
    first = jnp.logical_or(i == 0, tid_ref[i] != tid_ref[jnp.maximum(i - 1, 0)])

    @pl.when(first)
    def _():
        out_ref[...] = jnp.where(mask, h, jnp.zeros_like(h))

    @pl.when(jnp.logical_not(first))
    def _():
        out_ref[...] = jnp.where(mask, h, out_ref[...])


def _ffn2_body(tid_ref, gid_ref, rs_ref, re_ref, h_ref, w2_ref, ws_ref, out_ref):
    i = pl.program_id(0)
    dn = (((1,), (1,)), ((), ()))
    a = lax.dot_general(h_ref[...], w2_ref[0], dn, preferred_element_type=jnp.float32)
    a = a * ws_ref[...]
    rows = tid_ref[i] * BM + lax.broadcasted_iota(jnp.int32, (BM, 1), 0)
    mask = (rows >= rs_ref[i]) & (rows < re_ref[i])
    first = jnp.logical_or(i == 0, tid_ref[i] != tid_ref[jnp.maximum(i - 1, 0)])

    @pl.when(first)
    def _():
        out_ref[...] = jnp.where(mask, a, jnp.zeros_like(a))

    @pl.when(jnp.logical_not(first))
    def _():
        out_ref[...] = jnp.where(mask, a, out_ref[...])


def _ffn_fused_body(tid_ref, gid_ref, rs_ref, re_ref, x_ref, w1_ref, w3_ref,
                    w2_ref, ws_ref, out_ref):
    i = pl.program_id(0)
    dn = (((1,), (1,)), ((), ()))
    xb = x_ref[...].astype(jnp.bfloat16)
    a1 = lax.dot_general(xb, w1_ref[0], dn, preferred_element_type=jnp.float32)
    a3 = lax.dot_general(xb, w3_ref[0], dn, preferred_element_type=jnp.float32)
    h = (a1 * lax.logistic(a1) * a3).astype(jnp.bfloat16)
    d = lax.dot_general(h, w2_ref[0], dn, preferred_element_type=jnp.float32)
    d = d * ws_ref[...]
    rows = tid_ref[i] * BM + lax.broadcasted_iota(jnp.int32, (BM, 1), 0)
    mask = (rows >= rs_ref[i]) & (rows < re_ref[i])
    first = jnp.logical_or(i == 0, tid_ref[i] != tid_ref[jnp.maximum(i - 1, 0)])

    @pl.when(first)
    def _():
        out_ref[...] = jnp.where(mask, d, jnp.zeros_like(d))

    @pl.when(jnp.logical_not(first))
    def _():
        out_ref[...] = jnp.where(mask, d, out_ref[...])


def _grouped_ffn_fused(tid, gid, rs, re, xp, w1, w3, w2, ws):
    grid_spec = pltpu.PrefetchScalarGridSpec(
        num_scalar_prefetch=4,
        grid=(MAX_STEPS,),
        in_specs=[
            pl.BlockSpec((BM, DIM), lambda i, t, g, s, e: (t[i], 0)),
            pl.BlockSpec((1, INTER, DIM), lambda i, t, g, s, e: (g[i], 0, 0)),
            pl.BlockSpec((1, INTER, DIM), lambda i, t, g, s, e: (g[i], 0, 0)),
            pl.BlockSpec((1, DIM, INTER), lambda i, t, g, s, e: (g[i], 0, 0)),
            pl.BlockSpec((BM, 1), lambda i, t, g, s, e: (t[i], 0)),
        ],
        out_specs=pl.BlockSpec((BM, DIM), lambda i, t, g, s, e: (t[i], 0)),
    )
    return pl.pallas_call(
        _ffn_fused_body,
        grid_spec=grid_spec,
        out_shape=jax.ShapeDtypeStruct((RP, DIM), jnp.float32),
        compiler_params=pltpu.CompilerParams(
            dimension_semantics=("arbitrary",)),
    )(tid, gid, rs, re, xp, w1, w3, w2, ws)


def _grouped_ffn1(tid, gid, rs, re, xp, w1, w3):
    grid_spec = pltpu.PrefetchScalarGridSpec(
        num_scalar_prefetch=4,
        grid=(MAX_STEPS,),
        in_specs=[
            pl.BlockSpec((BM, DIM), lambda i, t, g, s, e: (t[i], 0)),
            pl.BlockSpec((1, INTER, DIM), lambda i, t, g, s, e: (g[i], 0, 0)),
            pl.BlockSpec((1, INTER, DIM), lambda i, t, g, s, e: (g[i], 0, 0)),
        ],
        out_specs=pl.BlockSpec((BM, INTER), lambda i, t, g, s, e: (t[i], 0)),
    )
    return pl.pallas_call(
        _ffn1_body,
        grid_spec=grid_spec,
        out_shape=jax.ShapeDtypeStruct((RP, INTER), jnp.bfloat16),
        compiler_params=pltpu.CompilerParams(
            dimension_semantics=("arbitrary",)),
    )(tid, gid, rs, re, xp, w1, w3)


def _grouped_ffn2(tid, gid, rs, re, up, w2, ws):
    grid_spec = pltpu.PrefetchScalarGridSpec(
        num_scalar_prefetch=4,
        grid=(MAX_STEPS,),
        in_specs=[
            pl.BlockSpec((BM, INTER), lambda i, t, g, s, e: (t[i], 0)),
            pl.BlockSpec((1, DIM, INTER), lambda i, t, g, s, e: (g[i], 0, 0)),
            pl.BlockSpec((BM, 1), lambda i, t, g, s, e: (t[i], 0)),
        ],
        out_specs=pl.BlockSpec((BM, DIM), lambda i, t, g, s, e: (t[i], 0)),
    )
    return pl.pallas_call(
        _ffn2_body,
        grid_spec=grid_spec,
        out_shape=jax.ShapeDtypeStruct((RP, DIM), jnp.float32),
        compiler_params=pltpu.CompilerParams(
            dimension_semantics=("arbitrary",)),
    )(tid, gid, rs, re, up, w2, ws)


def _routing_metadata(expert_indices):
    flat = expert_indices.reshape(-1).astype(jnp.int32)
    sorted_idx = jnp.argsort(flat).astype(jnp.int32)
    gather_idx = sorted_idx // TOPK
    counts = jnp.bincount(flat, length=NUM_EXPERTS).astype(jnp.int32)
    ends = jnp.cumsum(counts)
    starts = ends - counts
    t_lo = starts // BM
    t_hi = (ends + BM - 1) // BM
    gt = jnp.where(counts > 0, t_hi - t_lo, 0)
    steps = jnp.arange(MAX_STEPS, dtype=jnp.int32)
    gid = jnp.repeat(jnp.arange(NUM_EXPERTS, dtype=jnp.int32), gt,
                     total_repeat_length=MAX_STEPS)
    cum_gt = jnp.cumsum(gt) - gt
    num_steps = jnp.sum(gt)
    tid_raw = t_lo[gid] + steps - cum_gt[gid]
    valid = steps < num_steps
    tid = jnp.where(valid, tid_raw, NUM_TILES - 1).astype(jnp.int32)
    rs = jnp.where(valid, jnp.maximum(starts[gid], tid_raw * BM), 0).astype(jnp.int32)
    re = jnp.where(valid, jnp.minimum(ends[gid], (tid_raw + 1) * BM), 0).astype(jnp.int32)
    inv = jnp.zeros((RP,), jnp.int32).at[sorted_idx].set(
        jnp.arange(RP, dtype=jnp.int32))
    p0 = inv[0::2]
    p1 = inv[1::2]
    return sorted_idx, gather_idx, tid, gid, rs, re, p0, p1


def kernel(x, expert_indices, expert_weights, w1, w2, w3):
    sorted_idx, gather_idx, tid, gid, rs, re, p0, p1 = _routing_metadata(
        expert_indices)
    ws_sorted = expert_weights.reshape(-1)[sorted_idx].reshape(RP, 1)

    w1b = w1.astype(jnp.bfloat16)
    w3b = w3.astype(jnp.bfloat16)
    w2b = w2.astype(jnp.bfloat16)

    xp = _sc_gather(gather_idx, x)
    down = _grouped_ffn_fused(tid, gid, rs, re, xp, w1b, w3b, w2b, ws_sorted)
    out = _sc_combine(p0.reshape(T // C_CH, C_CH),
                      p1.reshape(T // C_CH, C_CH), down)
    return out


# speedup vs baseline: 1.4661x; 1.0004x over previous
import functools

import jax
import jax.numpy as jnp
from jax import lax
from jax.experimental import pallas as pl
from jax.experimental.pallas import tpu as pltpu
from jax.experimental.pallas import tpu_sc as plsc

NUM_EXPERTS = 16
TOPK = 2
DIM = 2048
INTER = 1664
T = 4096
RP = T * TOPK

BM = 256
NUM_TILES = RP // BM
MAX_STEPS = NUM_TILES + NUM_EXPERTS - 1

NC = 2
NS = 16
NW = NC * NS
G_ROWS_PER_W = RP // NW
G_CH = 16
G_NCH = G_ROWS_PER_W // G_CH
C_TOK_PER_W = T // NW
C_CH = 8
C_NCH = C_TOK_PER_W // C_CH


def _sc_gather(idx, table):
    mesh = plsc.VectorSubcoreMesh(core_axis_name="c", subcore_axis_name="s")

    @functools.partial(
        pl.kernel,
        mesh=mesh,
        out_type=jax.ShapeDtypeStruct((RP, DIM), jnp.float32),
        scratch_types=[
            pltpu.VMEM((G_ROWS_PER_W,), jnp.int32),
            pltpu.VMEM((2, G_CH, DIM), jnp.float32),
            pltpu.SemaphoreType.DMA,
            pltpu.SemaphoreType.DMA,
            pltpu.SemaphoreType.DMA,
            pltpu.SemaphoreType.DMA,
        ],
    )
    def gather_kernel(idx_hbm, table_hbm, out_hbm, idx_v, rows_v,
                      gs0, gs1, ws0, ws1):
        wid = lax.axis_index("s") * NC + lax.axis_index("c")
        base = wid * G_ROWS_PER_W
        pltpu.sync_copy(idx_hbm.at[pl.ds(base, G_ROWS_PER_W)], idx_v)
        gsems = (gs0, gs1)
        wsems = (ws0, ws1)
        gh = [None, None]
        wh = [None, None]
        for c in range(G_NCH):
            s = c & 1
            if c >= 2:
                wh[s].wait()
            ids = idx_v[pl.ds(c * G_CH, G_CH)]
            gh[s] = pltpu.async_copy(table_hbm.at[ids], rows_v.at[s], gsems[s])
            if c >= 1:
                p = (c - 1) & 1
                gh[p].wait()
                wh[p] = pltpu.async_copy(
                    rows_v.at[p],
                    out_hbm.at[pl.ds(base + (c - 1) * G_CH, G_CH)], wsems[p])
        lastp = (G_NCH - 1) & 1
        gh[lastp].wait()
        wh[lastp] = pltpu.async_copy(
            rows_v.at[lastp],
            out_hbm.at[pl.ds(base + (G_NCH - 1) * G_CH, G_CH)], wsems[lastp])
        wh[1 - lastp].wait()
        wh[lastp].wait()

    return gather_kernel(idx, table)


def _sc_combine(p0, p1, rows):
    mesh = plsc.VectorSubcoreMesh(core_axis_name="c", subcore_axis_name="s")

    @functools.partial(
        pl.kernel,
        mesh=mesh,
        out_type=jax.ShapeDtypeStruct((T, DIM), jnp.float32),
        scratch_types=[
            pltpu.VMEM((C_NCH, C_CH), jnp.int32),
            pltpu.VMEM((C_NCH, C_CH), jnp.int32),
            pltpu.VMEM((2, C_CH, DIM), jnp.float32),
            pltpu.VMEM((2, C_CH, DIM), jnp.float32),
            pltpu.SemaphoreType.DMA,
            pltpu.SemaphoreType.DMA,
            pltpu.SemaphoreType.DMA,
            pltpu.SemaphoreType.DMA,
            pltpu.SemaphoreType.DMA,
            pltpu.SemaphoreType.DMA,
        ],
    )
    def combine_kernel(p0_hbm, p1_hbm, rows_hbm, out_hbm, p0_v, p1_v,
                       bufa, bufb, ga0, ga1, gb0, gb1, ws0, ws1):
        wid = lax.axis_index("s") * NC + lax.axis_index("c")
        base = wid * C_TOK_PER_W
        cbase = wid * C_NCH
        pltpu.sync_copy(p0_hbm.at[pl.ds(cbase, C_NCH)], p0_v)
        pltpu.sync_copy(p1_hbm.at[pl.ds(cbase, C_NCH)], p1_v)
        gasems = (ga0, ga1)
        gbsems = (gb0, gb1)
        wsems = (ws0, ws1)
        gha = [None, None]
        ghb = [None, None]
        wh = [None, None]

        def add_and_write(p, c):
            for r in range(C_CH):
                def add_body(l, cc, _r=r, _p=p):
                    for u in range(8):
                        o = (l * 8 + u) * 16
                        bufa[_p, _r, pl.ds(o, 16)] = (
                            bufa[_p, _r, pl.ds(o, 16)]
                            + bufb[_p, _r, pl.ds(o, 16)])
                    return cc
                lax.fori_loop(0, DIM // (8 * 16), add_body, 0)
            return pltpu.async_copy(
                bufa.at[p], out_hbm.at[pl.ds(base + c * C_CH, C_CH)], wsems[p])

        for c in range(C_NCH):
            s = c & 1
            if c >= 2:
                wh[s].wait()
            gha[s] = pltpu.async_copy(rows_hbm.at[p0_v.at[c]], bufa.at[s],
                                      gasems[s])
            ghb[s] = pltpu.async_copy(rows_hbm.at[p1_v.at[c]], bufb.at[s],
                                      gbsems[s])
            if c >= 1:
                p = (c - 1) & 1
                gha[p].wait()
                ghb[p].wait()
                wh[p] = add_and_write(p, c - 1)
        lastp = (C_NCH - 1) & 1
        gha[lastp].wait()
        ghb[lastp].wait()
        wh[lastp] = add_and_write(lastp, C_NCH - 1)
        wh[1 - lastp].wait()
        wh[lastp].wait()

    return combine_kernel(p0, p1, rows)


def _ffn1_body(tid_ref, gid_ref, rs_ref, re_ref, x_ref, w1_ref, w3_ref, out_ref):
    i = pl.program_id(0)
    dn = (((1,), (1,)), ((), ()))
    xb = x_ref[...].astype(jnp.bfloat16)
    a1 = lax.dot_general(xb, w1_ref[0], dn, preferred_element_type=jnp.float32)
    a3 = lax.dot_general(xb, w3_ref[0], dn, preferred_element_type=jnp.float32)
    h = (a1 * lax.logistic(a1) * a3).astype(jnp.bfloat16)
    rows = tid_ref[i] * BM + lax.broadcasted_iota(jnp.int32, (BM, 1), 0)
    mask = (rows >= rs_ref[i]) & (rows < re_ref[i])
    first = jnp.logical_or(i == 0, tid_ref[i] != tid_ref[jnp.maximum(i - 1, 0)])

    @pl.when(first)
    def _():
        out_ref[...] = jnp.where(mask, h, jnp.zeros_like(h))

    @pl.when(jnp.logical_not(first))
    def _():
        out_ref[...] = jnp.where(mask, h, out_ref[...])


def _ffn2_body(tid_ref, gid_ref, rs_ref, re_ref, h_ref, w2_ref, ws_ref, out_ref):
    i = pl.program_id(0)
    dn = (((1,), (1,)), ((), ()))
    a = lax.dot_general(h_ref[...], w2_ref[0], dn, preferred_element_type=jnp.float32)
    a = a * ws_ref[...]
    rows = tid_ref[i] * BM + lax.broadcasted_iota(jnp.int32, (BM, 1), 0)
    mask = (rows >= rs_ref[i]) & (rows < re_ref[i])
    first = jnp.logical_or(i == 0, tid_ref[i] != tid_ref[jnp.maximum(i - 1, 0)])

    @pl.when(first)
    def _():
        out_ref[...] = jnp.where(mask, a, jnp.zeros_like(a))

    @pl.when(jnp.logical_not(first))
    def _():
        out_ref[...] = jnp.where(mask, a, out_ref[...])


def _ffn_fused_body(tid_ref, gid_ref, rs_ref, re_ref, x_ref, w1_ref, w3_ref,
                    w2_ref, ws_ref, out_ref):
    i = pl.program_id(0)
    dn = (((1,), (1,)), ((), ()))
    xb = x_ref[...].astype(jnp.bfloat16)
    a1 = lax.dot_general(xb, w1_ref[0], dn, preferred_element_type=jnp.float32)
    a3 = lax.dot_general(xb, w3_ref[0], dn, preferred_element_type=jnp.float32)
    h = (a1 * lax.logistic(a1) * a3).astype(jnp.bfloat16)
    d = lax.dot_general(h, w2_ref[0], dn, preferred_element_type=jnp.float32)
    d = d * ws_ref[...]
    t = tid_ref[i]
    rows = t * BM + lax.broadcasted_iota(jnp.int32, (BM, 1), 0)
    mask = (rows >= rs_ref[i]) & (rows < re_ref[i])
    first = jnp.logical_or(i == 0, t != tid_ref[jnp.maximum(i - 1, 0)])
    full = jnp.logical_and(rs_ref[i] <= t * BM, re_ref[i] >= t * BM + BM)

    @pl.when(full)
    def _():
        out_ref[...] = d

    @pl.when(jnp.logical_and(jnp.logical_not(full), first))
    def _():
        out_ref[...] = jnp.where(mask, d, jnp.zeros_like(d))

    @pl.when(jnp.logical_and(jnp.logical_not(full), jnp.logical_not(first)))
    def _():
        out_ref[...] = jnp.where(mask, d, out_ref[...])


def _grouped_ffn_fused(tid, gid, rs, re, xp, w1, w3, w2, ws):
    grid_spec = pltpu.PrefetchScalarGridSpec(
        num_scalar_prefetch=4,
        grid=(MAX_STEPS,),
        in_specs=[
            pl.BlockSpec((BM, DIM), lambda i, t, g, s, e: (t[i], 0)),
            pl.BlockSpec((1, INTER, DIM), lambda i, t, g, s, e: (g[i], 0, 0)),
            pl.BlockSpec((1, INTER, DIM), lambda i, t, g, s, e: (g[i], 0, 0)),
            pl.BlockSpec((1, DIM, INTER), lambda i, t, g, s, e: (g[i], 0, 0)),
            pl.BlockSpec((BM, 1), lambda i, t, g, s, e: (t[i], 0)),
        ],
        out_specs=pl.BlockSpec((BM, DIM), lambda i, t, g, s, e: (t[i], 0)),
    )
    return pl.pallas_call(
        _ffn_fused_body,
        grid_spec=grid_spec,
        out_shape=jax.ShapeDtypeStruct((RP, DIM), jnp.float32),
        compiler_params=pltpu.CompilerParams(
            dimension_semantics=("arbitrary",)),
    )(tid, gid, rs, re, xp, w1, w3, w2, ws)


def _grouped_ffn1(tid, gid, rs, re, xp, w1, w3):
    grid_spec = pltpu.PrefetchScalarGridSpec(
        num_scalar_prefetch=4,
        grid=(MAX_STEPS,),
        in_specs=[
            pl.BlockSpec((BM, DIM), lambda i, t, g, s, e: (t[i], 0)),
            pl.BlockSpec((1, INTER, DIM), lambda i, t, g, s, e: (g[i], 0, 0)),
            pl.BlockSpec((1, INTER, DIM), lambda i, t, g, s, e: (g[i], 0, 0)),
        ],
        out_specs=pl.BlockSpec((BM, INTER), lambda i, t, g, s, e: (t[i], 0)),
    )
    return pl.pallas_call(
        _ffn1_body,
        grid_spec=grid_spec,
        out_shape=jax.ShapeDtypeStruct((RP, INTER), jnp.bfloat16),
        compiler_params=pltpu.CompilerParams(
            dimension_semantics=("arbitrary",)),
    )(tid, gid, rs, re, xp, w1, w3)


def _grouped_ffn2(tid, gid, rs, re, up, w2, ws):
    grid_spec = pltpu.PrefetchScalarGridSpec(
        num_scalar_prefetch=4,
        grid=(MAX_STEPS,),
        in_specs=[
            pl.BlockSpec((BM, INTER), lambda i, t, g, s, e: (t[i], 0)),
            pl.BlockSpec((1, DIM, INTER), lambda i, t, g, s, e: (g[i], 0, 0)),
            pl.BlockSpec((BM, 1), lambda i, t, g, s, e: (t[i], 0)),
        ],
        out_specs=pl.BlockSpec((BM, DIM), lambda i, t, g, s, e: (t[i], 0)),
    )
    return pl.pallas_call(
        _ffn2_body,
        grid_spec=grid_spec,
        out_shape=jax.ShapeDtypeStruct((RP, DIM), jnp.float32),
        compiler_params=pltpu.CompilerParams(
            dimension_semantics=("arbitrary",)),
    )(tid, gid, rs, re, up, w2, ws)


def _routing_metadata(expert_indices):
    flat = expert_indices.reshape(-1).astype(jnp.int32)
    sorted_idx = jnp.argsort(flat).astype(jnp.int32)
    gather_idx = sorted_idx // TOPK
    counts = jnp.bincount(flat, length=NUM_EXPERTS).astype(jnp.int32)
    ends = jnp.cumsum(counts)
    starts = ends - counts
    t_lo = starts // BM
    t_hi = (ends + BM - 1) // BM
    gt = jnp.where(counts > 0, t_hi - t_lo, 0)
    steps = jnp.arange(MAX_STEPS, dtype=jnp.int32)
    gid = jnp.repeat(jnp.arange(NUM_EXPERTS, dtype=jnp.int32), gt,
                     total_repeat_length=MAX_STEPS)
    cum_gt = jnp.cumsum(gt) - gt
    num_steps = jnp.sum(gt)
    tid_raw = t_lo[gid] + steps - cum_gt[gid]
    valid = steps < num_steps
    tid = jnp.where(valid, tid_raw, NUM_TILES - 1).astype(jnp.int32)
    rs = jnp.where(valid, jnp.maximum(starts[gid], tid_raw * BM), 0).astype(jnp.int32)
    re = jnp.where(valid, jnp.minimum(ends[gid], (tid_raw + 1) * BM), 0).astype(jnp.int32)
    inv = jnp.zeros((RP,), jnp.int32).at[sorted_idx].set(
        jnp.arange(RP, dtype=jnp.int32))
    p0 = inv[0::2]
    p1 = inv[1::2]
    return sorted_idx, gather_idx, tid, gid, rs, re, p0, p1


def kernel(x, expert_indices, expert_weights, w1, w2, w3):
    sorted_idx, gather_idx, tid, gid, rs, re, p0, p1 = _routing_metadata(
        expert_indices)
    ws_sorted = expert_weights.reshape(-1)[sorted_idx].reshape(RP, 1)

    w1b = w1.astype(jnp.bfloat16)
    w3b = w3.astype(jnp.bfloat16)
    w2b = w2.astype(jnp.bfloat16)

    xp = _sc_gather(gather_idx, x)
    down = _grouped_ffn_fused(tid, gid, rs, re, xp, w1b, w3b, w2b, ws_sorted)
    out = _sc_combine(p0.reshape(T // C_CH, C_CH),
                      p1.reshape(T // C_CH, C_CH), down)
    return out


# speedup vs baseline: 1.4977x; 1.0216x over previous
import functools

import jax
import jax.numpy as jnp
from jax import lax
from jax.experimental import pallas as pl
from jax.experimental.pallas import tpu as pltpu
from jax.experimental.pallas import tpu_sc as plsc

NUM_EXPERTS = 16
TOPK = 2
DIM = 2048
INTER = 1664
T = 4096
RP = T * TOPK

BM = 256
NUM_TILES = RP // BM
MAX_STEPS = NUM_TILES + NUM_EXPERTS - 1

NC = 2
NS = 16
NW = NC * NS
G_ROWS_PER_W = RP // NW
G_CH = 16
G_NCH = G_ROWS_PER_W // G_CH
C_TOK_PER_W = T // NW
C_CH = 8
C_NCH = C_TOK_PER_W // C_CH


def _sc_gather(idx, table):
    mesh = plsc.VectorSubcoreMesh(core_axis_name="c", subcore_axis_name="s")

    @functools.partial(
        pl.kernel,
        mesh=mesh,
        out_type=jax.ShapeDtypeStruct((RP, DIM), jnp.float32),
        scratch_types=[
            pltpu.VMEM((G_ROWS_PER_W,), jnp.int32),
            pltpu.VMEM((2, G_CH, DIM), jnp.float32),
            pltpu.SemaphoreType.DMA,
            pltpu.SemaphoreType.DMA,
            pltpu.SemaphoreType.DMA,
            pltpu.SemaphoreType.DMA,
        ],
    )
    def gather_kernel(idx_hbm, table_hbm, out_hbm, idx_v, rows_v,
                      gs0, gs1, ws0, ws1):
        wid = lax.axis_index("s") * NC + lax.axis_index("c")
        base = wid * G_ROWS_PER_W
        pltpu.sync_copy(idx_hbm.at[pl.ds(base, G_ROWS_PER_W)], idx_v)
        gsems = (gs0, gs1)
        wsems = (ws0, ws1)
        gh = [None, None]
        wh = [None, None]
        for c in range(G_NCH):
            s = c & 1
            if c >= 2:
                wh[s].wait()
            ids = idx_v[pl.ds(c * G_CH, G_CH)]
            gh[s] = pltpu.async_copy(table_hbm.at[ids], rows_v.at[s], gsems[s])
            if c >= 1:
                p = (c - 1) & 1
                gh[p].wait()
                wh[p] = pltpu.async_copy(
                    rows_v.at[p],
                    out_hbm.at[pl.ds(base + (c - 1) * G_CH, G_CH)], wsems[p])
        lastp = (G_NCH - 1) & 1
        gh[lastp].wait()
        wh[lastp] = pltpu.async_copy(
            rows_v.at[lastp],
            out_hbm.at[pl.ds(base + (G_NCH - 1) * G_CH, G_CH)], wsems[lastp])
        wh[1 - lastp].wait()
        wh[lastp].wait()

    return gather_kernel(idx, table)


def _sc_combine(p0, p1, rows):
    mesh = plsc.VectorSubcoreMesh(core_axis_name="c", subcore_axis_name="s")

    @functools.partial(
        pl.kernel,
        mesh=mesh,
        out_type=jax.ShapeDtypeStruct((T, DIM), jnp.float32),
        scratch_types=[
            pltpu.VMEM((C_NCH, C_CH), jnp.int32),
            pltpu.VMEM((C_NCH, C_CH), jnp.int32),
            pltpu.VMEM((2, C_CH, DIM), jnp.float32),
            pltpu.VMEM((2, C_CH, DIM), jnp.float32),
            pltpu.SemaphoreType.DMA,
            pltpu.SemaphoreType.DMA,
            pltpu.SemaphoreType.DMA,
            pltpu.SemaphoreType.DMA,
            pltpu.SemaphoreType.DMA,
            pltpu.SemaphoreType.DMA,
        ],
    )
    def combine_kernel(p0_hbm, p1_hbm, rows_hbm, out_hbm, p0_v, p1_v,
                       bufa, bufb, ga0, ga1, gb0, gb1, ws0, ws1):
        wid = lax.axis_index("s") * NC + lax.axis_index("c")
        base = wid * C_TOK_PER_W
        cbase = wid * C_NCH
        pltpu.sync_copy(p0_hbm.at[pl.ds(cbase, C_NCH)], p0_v)
        pltpu.sync_copy(p1_hbm.at[pl.ds(cbase, C_NCH)], p1_v)
        gasems = (ga0, ga1)
        gbsems = (gb0, gb1)
        wsems = (ws0, ws1)
        gha = [None, None]
        ghb = [None, None]
        wh = [None, None]

        def add_and_write(p, c):
            for r in range(C_CH):
                def add_body(l, cc, _r=r, _p=p):
                    for u in range(8):
                        o = (l * 8 + u) * 16
                        bufa[_p, _r, pl.ds(o, 16)] = (
                            bufa[_p, _r, pl.ds(o, 16)]
                            + bufb[_p, _r, pl.ds(o, 16)])
                    return cc
                lax.fori_loop(0, DIM // (8 * 16), add_body, 0)
            return pltpu.async_copy(
                bufa.at[p], out_hbm.at[pl.ds(base + c * C_CH, C_CH)], wsems[p])

        for c in range(C_NCH):
            s = c & 1
            if c >= 2:
                wh[s].wait()
            gha[s] = pltpu.async_copy(rows_hbm.at[p0_v.at[c]], bufa.at[s],
                                      gasems[s])
            ghb[s] = pltpu.async_copy(rows_hbm.at[p1_v.at[c]], bufb.at[s],
                                      gbsems[s])
            if c >= 1:
                p = (c - 1) & 1
                gha[p].wait()
                ghb[p].wait()
                wh[p] = add_and_write(p, c - 1)
        lastp = (C_NCH - 1) & 1
        gha[lastp].wait()
        ghb[lastp].wait()
        wh[lastp] = add_and_write(lastp, C_NCH - 1)
        wh[1 - lastp].wait()
        wh[lastp].wait()

    return combine_kernel(p0, p1, rows)


def _ffn1_body(tid_ref, gid_ref, rs_ref, re_ref, x_ref, w1_ref, w3_ref, out_ref,
               acc1, acc3):
    i = pl.program_id(0)
    k = pl.program_id(1)
    dn = (((1,), (1,)), ((), ()))
    xb = x_ref[...]
    p1 = lax.dot_general(xb, w1_ref[0], dn, preferred_element_type=jnp.float32,
                         precision=lax.Precision.DEFAULT)
    p3 = lax.dot_general(xb, w3_ref[0], dn, preferred_element_type=jnp.float32,
                         precision=lax.Precision.DEFAULT)

    @pl.when(k == 0)
    def _():
        acc1[...] = p1
        acc3[...] = p3

    @pl.when(k == 1)
    def _():
        a1 = acc1[...] + p1
        a3 = acc3[...] + p3
        h = (a1 * lax.logistic(a1) * a3).astype(jnp.bfloat16)
        t = tid_ref[i]
        rows = t * BM + lax.broadcasted_iota(jnp.int32, (BM, 1), 0)
        mask = (rows >= rs_ref[i]) & (rows < re_ref[i])
        first = jnp.logical_or(i == 0, t != tid_ref[jnp.maximum(i - 1, 0)])
        full = jnp.logical_and(rs_ref[i] <= t * BM, re_ref[i] >= t * BM + BM)

        @pl.when(full)
        def _():
            out_ref[...] = h

        @pl.when(jnp.logical_and(jnp.logical_not(full), first))
        def _():
            out_ref[...] = jnp.where(mask, h, jnp.zeros_like(h))

        @pl.when(jnp.logical_and(jnp.logical_not(full), jnp.logical_not(first)))
        def _():
            out_ref[...] = jnp.where(mask, h, out_ref[...])


def _ffn2_body(tid_ref, gid_ref, rs_ref, re_ref, h_ref, w2_ref, ws_ref, out_ref):
    i = pl.program_id(0)
    dn = (((1,), (1,)), ((), ()))
    a = lax.dot_general(h_ref[...].astype(jnp.float32), w2_ref[0], dn,
                        preferred_element_type=jnp.float32,
                        precision=lax.Precision.DEFAULT)
    a = a * ws_ref[...]
    t = tid_ref[i]
    rows = t * BM + lax.broadcasted_iota(jnp.int32, (BM, 1), 0)
    mask = (rows >= rs_ref[i]) & (rows < re_ref[i])
    first = jnp.logical_or(i == 0, t != tid_ref[jnp.maximum(i - 1, 0)])
    full = jnp.logical_and(rs_ref[i] <= t * BM, re_ref[i] >= t * BM + BM)

    @pl.when(full)
    def _():
        out_ref[...] = a

    @pl.when(jnp.logical_and(jnp.logical_not(full), first))
    def _():
        out_ref[...] = jnp.where(mask, a, jnp.zeros_like(a))

    @pl.when(jnp.logical_and(jnp.logical_not(full), jnp.logical_not(first)))
    def _():
        out_ref[...] = jnp.where(mask, a, out_ref[...])


def _ffn_fused_body(tid_ref, gid_ref, rs_ref, re_ref, x_ref, w1_ref, w3_ref,
                    w2_ref, ws_ref, out_ref):
    i = pl.program_id(0)
    dn = (((1,), (1,)), ((), ()))
    xb = x_ref[...].astype(jnp.bfloat16)
    a1 = lax.dot_general(xb, w1_ref[0], dn, preferred_element_type=jnp.float32)
    a3 = lax.dot_general(xb, w3_ref[0], dn, preferred_element_type=jnp.float32)
    h = (a1 * lax.logistic(a1) * a3).astype(jnp.bfloat16)
    d = lax.dot_general(h, w2_ref[0], dn, preferred_element_type=jnp.float32)
    d = d * ws_ref[...]
    t = tid_ref[i]
    rows = t * BM + lax.broadcasted_iota(jnp.int32, (BM, 1), 0)
    mask = (rows >= rs_ref[i]) & (rows < re_ref[i])
    first = jnp.logical_or(i == 0, t != tid_ref[jnp.maximum(i - 1, 0)])
    full = jnp.logical_and(rs_ref[i] <= t * BM, re_ref[i] >= t * BM + BM)

    @pl.when(full)
    def _():
        out_ref[...] = d

    @pl.when(jnp.logical_and(jnp.logical_not(full), first))
    def _():
        out_ref[...] = jnp.where(mask, d, jnp.zeros_like(d))

    @pl.when(jnp.logical_and(jnp.logical_not(full), jnp.logical_not(first)))
    def _():
        out_ref[...] = jnp.where(mask, d, out_ref[...])


def _grouped_ffn_fused(tid, gid, rs, re, xp, w1, w3, w2, ws):
    grid_spec = pltpu.PrefetchScalarGridSpec(
        num_scalar_prefetch=4,
        grid=(MAX_STEPS,),
        in_specs=[
            pl.BlockSpec((BM, DIM), lambda i, t, g, s, e: (t[i], 0)),
            pl.BlockSpec((1, INTER, DIM), lambda i, t, g, s, e: (g[i], 0, 0)),
            pl.BlockSpec((1, INTER, DIM), lambda i, t, g, s, e: (g[i], 0, 0)),
            pl.BlockSpec((1, DIM, INTER), lambda i, t, g, s, e: (g[i], 0, 0)),
            pl.BlockSpec((BM, 1), lambda i, t, g, s, e: (t[i], 0)),
        ],
        out_specs=pl.BlockSpec((BM, DIM), lambda i, t, g, s, e: (t[i], 0)),
    )
    return pl.pallas_call(
        _ffn_fused_body,
        grid_spec=grid_spec,
        out_shape=jax.ShapeDtypeStruct((RP, DIM), jnp.float32),
        compiler_params=pltpu.CompilerParams(
            dimension_semantics=("arbitrary",)),
    )(tid, gid, rs, re, xp, w1, w3, w2, ws)


def _grouped_ffn1(tid, gid, rs, re, xp, w1, w3):
    grid_spec = pltpu.PrefetchScalarGridSpec(
        num_scalar_prefetch=4,
        grid=(MAX_STEPS, 2),
        in_specs=[
            pl.BlockSpec((BM, DIM // 2), lambda i, k, t, g, s, e: (t[i], k)),
            pl.BlockSpec((1, INTER, DIM // 2),
                         lambda i, k, t, g, s, e: (g[i], 0, k)),
            pl.BlockSpec((1, INTER, DIM // 2),
                         lambda i, k, t, g, s, e: (g[i], 0, k)),
        ],
        out_specs=pl.BlockSpec((BM, INTER), lambda i, k, t, g, s, e: (t[i], 0)),
        scratch_shapes=[
            pltpu.VMEM((BM, INTER), jnp.float32),
            pltpu.VMEM((BM, INTER), jnp.float32),
        ],
    )
    return pl.pallas_call(
        _ffn1_body,
        grid_spec=grid_spec,
        out_shape=jax.ShapeDtypeStruct((RP, INTER), jnp.bfloat16),
        compiler_params=pltpu.CompilerParams(
            dimension_semantics=("arbitrary", "arbitrary")),
    )(tid, gid, rs, re, xp, w1, w3)


def _grouped_ffn2(tid, gid, rs, re, up, w2, ws):
    grid_spec = pltpu.PrefetchScalarGridSpec(
        num_scalar_prefetch=4,
        grid=(MAX_STEPS,),
        in_specs=[
            pl.BlockSpec((BM, INTER), lambda i, t, g, s, e: (t[i], 0)),
            pl.BlockSpec((1, DIM, INTER), lambda i, t, g, s, e: (g[i], 0, 0)),
            pl.BlockSpec((BM, 1), lambda i, t, g, s, e: (t[i], 0)),
        ],
        out_specs=pl.BlockSpec((BM, DIM), lambda i, t, g, s, e: (t[i], 0)),
    )
    return pl.pallas_call(
        _ffn2_body,
        grid_spec=grid_spec,
        out_shape=jax.ShapeDtypeStruct((RP, DIM), jnp.float32),
        compiler_params=pltpu.CompilerParams(
            dimension_semantics=("arbitrary",)),
    )(tid, gid, rs, re, up, w2, ws)


def _routing_metadata(expert_indices):
    flat = expert_indices.reshape(-1).astype(jnp.int32)
    sorted_idx = jnp.argsort(flat).astype(jnp.int32)
    gather_idx = sorted_idx // TOPK
    counts = jnp.bincount(flat, length=NUM_EXPERTS).astype(jnp.int32)
    ends = jnp.cumsum(counts)
    starts = ends - counts
    t_lo = starts // BM
    t_hi = (ends + BM - 1) // BM
    gt = jnp.where(counts > 0, t_hi - t_lo, 0)
    steps = jnp.arange(MAX_STEPS, dtype=jnp.int32)
    gid = jnp.repeat(jnp.arange(NUM_EXPERTS, dtype=jnp.int32), gt,
                     total_repeat_length=MAX_STEPS)
    cum_gt = jnp.cumsum(gt) - gt
    num_steps = jnp.sum(gt)
    tid_raw = t_lo[gid] + steps - cum_gt[gid]
    valid = steps < num_steps
    tid = jnp.where(valid, tid_raw, NUM_TILES - 1).astype(jnp.int32)
    rs = jnp.where(valid, jnp.maximum(starts[gid], tid_raw * BM), 0).astype(jnp.int32)
    re = jnp.where(valid, jnp.minimum(ends[gid], (tid_raw + 1) * BM), 0).astype(jnp.int32)
    inv = jnp.zeros((RP,), jnp.int32).at[sorted_idx].set(
        jnp.arange(RP, dtype=jnp.int32))
    p0 = inv[0::2]
    p1 = inv[1::2]
    return sorted_idx, gather_idx, tid, gid, rs, re, p0, p1


def kernel(x, expert_indices, expert_weights, w1, w2, w3):
    sorted_idx, gather_idx, tid, gid, rs, re, p0, p1 = _routing_metadata(
        expert_indices)
    ws_sorted = expert_weights.reshape(-1)[sorted_idx].reshape(RP, 1)

    xp = _sc_gather(gather_idx, x)
    up = _grouped_ffn1(tid, gid, rs, re, xp, w1, w3)
    down = _grouped_ffn2(tid, gid, rs, re, up, w2, ws_sorted)
    out = _sc_combine(p0.reshape(T // C_CH, C_CH),
                      p1.reshape(T // C_CH, C_CH), down)
    return out
